# Initial kernel scaffold; baseline (speedup 1.0000x reference)
import jax
import jax.numpy as jnp
from jax.experimental import pallas as pl


def kernel(dynamic_emb, emb_rel, W_ih, b_ih, W_hh, b_hh, W_agg, W_t, b_t, conv_w, conv_b, fc_w, fc_b, sentence_embeddings, edge_src, edge_dst, edge_type, sub, rel, obj):
    raise NotImplementedError("write your pallas kernel here")


# speedup vs baseline: 1.0009x; 1.0009x over previous
import jax
import jax.numpy as jnp
from jax.experimental import pallas as pl

NUM_ENTS = 10000
NUM_RELS = 100
H = 128
E = 320000
CCH = 32
KW = 3


def _l2norm(x):
    return x / jnp.maximum(jnp.linalg.norm(x, axis=-1, keepdims=True), 1e-12)


def _entnorm_body(x_ref, o_ref):
    x = x_ref[...]
    n = jnp.maximum(jnp.sqrt(jnp.sum(x * x, axis=-1, keepdims=True)), 1e-12)
    o_ref[...] = x / n


def kernel(dynamic_emb, emb_rel, W_ih, b_ih, W_hh, b_hh, W_agg, W_t, b_t,
           conv_w, conv_b, fc_w, fc_b, sentence_embeddings,
           edge_src, edge_dst, edge_type, sub, rel, obj):
    ent = pl.pallas_call(
        _entnorm_body,
        out_shape=jax.ShapeDtypeStruct((NUM_ENTS, H), jnp.float32),
    )(dynamic_emb[0])
    rel_emb = emb_rel[0]
    temp_e = jnp.take(ent, edge_src, axis=0)
    seg_sum = jax.ops.segment_sum(temp_e, edge_type, num_segments=2 * NUM_RELS)
    cnt = jax.ops.segment_sum(jnp.ones((E,), jnp.float32), edge_type,
                              num_segments=2 * NUM_RELS)
    x_mean = seg_sum / jnp.maximum(cnt, 1.0)[:, None]
    x_input = jnp.concatenate([rel_emb, x_mean], axis=1)
    gi = x_input @ W_ih.T + b_ih
    gh = rel_emb @ W_hh.T + b_hh
    i_r, i_z, i_n = jnp.split(gi, 3, axis=1)
    h_r, h_z, h_n = jnp.split(gh, 3, axis=1)
    r_g = jax.nn.sigmoid(i_r + h_r)
    z_g = jax.nn.sigmoid(i_z + h_z)
    n_g = jnp.tanh(i_n + r_g * h_n)
    rel_new = (1.0 - z_g) * n_g + z_g * rel_emb
    rel_new = _l2norm(rel_new)
    msg = jnp.take(ent, edge_src, axis=0) + jnp.take(rel_new, edge_type, axis=0)
    agg = jax.ops.segment_sum(msg, edge_dst, num_segments=NUM_ENTS)
    deg = jax.ops.segment_sum(jnp.ones((E,), jnp.float32), edge_dst,
                              num_segments=NUM_ENTS)
    agg = agg / jnp.maximum(deg, 1.0)[:, None]
    curr = jax.nn.relu(agg @ W_agg)
    curr = _l2norm(curr)
    tw = jax.nn.sigmoid(ent @ W_t + b_t)
    ent_new = tw * curr + (1.0 - tw) * ent
    pre_emb = _l2norm(ent_new)
    all_sub = jnp.concatenate([sub, obj])
    all_rel = jnp.concatenate([rel, rel + NUM_RELS])
    all_obj = jnp.concatenate([obj, sub])
    sent = jnp.concatenate([sentence_embeddings, sentence_embeddings], axis=0)
    e_s = jnp.take(pre_emb, all_sub, axis=0)
    r_e = jnp.take(rel_new, all_rel, axis=0)
    x = jnp.stack([e_s, r_e, sent], axis=1)
    y = jax.lax.conv_general_dilated(x, conv_w, (1,), 'SAME',
                                     dimension_numbers=('NCH', 'OIH', 'NCH'))
    y = jax.nn.relu(y + conv_b[None, :, None])
    feat = jax.nn.relu(y.reshape(y.shape[0], -1) @ fc_w + fc_b)
    scores = feat @ pre_emb.T
    logp = jax.nn.log_softmax(scores, axis=-1)
    loss = -jnp.mean(jnp.take_along_axis(logp, all_obj[:, None], axis=1))
    return loss


# speedup vs baseline: 4.3334x; 4.3296x over previous
import jax
import jax.numpy as jnp
from jax import lax
from jax.experimental import pallas as pl
from jax.experimental.pallas import tpu as pltpu
from jax.experimental.pallas import tpu_sc as plsc

NE = 10000
NR = 100
NT = 200
H = 128
E = 320000
B2 = 2048
CCH = 32

NC = 2
NS = 16
NW = NC * NS

CH = 128
PAD_E = 323584
CTOT = 1024000
CPT = CTOT // NS
ZB = 8000
DROWS = 10240
DPT = DROWS // NS
S2_ROWS = 10240
S2_RPT = S2_ROWS // NS

_sc_mesh = plsc.VectorSubcoreMesh(core_axis_name="c", subcore_axis_name="s")


def _sc_kernel(body, out_type, scratch_types):
    return pl.kernel(body, out_type=out_type, mesh=_sc_mesh,
                     scratch_types=scratch_types)


def _sc_counts_body(typ_hbm, src_hbm, dst_hbm, zeros_hbm,
                    c1a_hbm, c1b_hbm, deg_hbm, typ_v, src_v, dst_v, fidx_v,
                    ones_v, zb_v, acc, deg_acc):
    c = lax.axis_index("c")
    s = lax.axis_index("s")
    pltpu.sync_copy(zeros_hbm, zb_v)

    def zbody(i, _):
        pltpu.sync_copy(zb_v, acc.at[pl.ds(s * CPT + i * ZB, ZB)])
        return ()

    lax.fori_loop(0, CPT // ZB, zbody, (), unroll=False)

    @pl.when(c == 0)
    def _():
        pltpu.sync_copy(zb_v.at[pl.ds(0, DPT)], deg_acc.at[pl.ds(s * DPT, DPT)])

    for j in range(CH // 16):
        ones_v[pl.ds(j * 16, 16)] = jnp.ones((16,), jnp.float32)
    plsc.subcore_barrier()

    n_chunks = PAD_E // NS // CH
    base_t = c * NR * NE

    def body(i, _):
        base = (s * n_chunks + i) * CH
        pltpu.sync_copy(typ_hbm.at[pl.ds(base, CH)], typ_v)
        pltpu.sync_copy(src_hbm.at[pl.ds(base, CH)], src_v)
        for j in range(CH // 16):
            t = typ_v[pl.ds(j * 16, 16)]
            sr = src_v[pl.ds(j * 16, 16)]
            loc = t * NE + sr - base_t
            valid = (loc >= 0) & (loc < NR * NE)
            fidx_v[pl.ds(j * 16, 16)] = jnp.where(valid, loc, NR * NE + sr)
        pltpu.sync_copy(ones_v, acc.at[fidx_v], add=True)

        @pl.when(c == 0)
        def _():
            pltpu.sync_copy(dst_hbm.at[pl.ds(base, CH)], dst_v)
            pltpu.sync_copy(ones_v, deg_acc.at[dst_v], add=True)

        return ()

    lax.fori_loop(0, n_chunks, body, (), unroll=False)
    plsc.subcore_barrier()

    def wbody(i, _):
        off = s * CPT + i * ZB
        pltpu.sync_copy(acc.at[pl.ds(off, ZB)], zb_v)

        @pl.when(c == 0)
        def _():
            pltpu.sync_copy(zb_v, c1a_hbm.at[pl.ds(off, ZB)])

        @pl.when(c == 1)
        def _():
            pltpu.sync_copy(zb_v, c1b_hbm.at[pl.ds(off, ZB)])

        return ()

    lax.fori_loop(0, CPT // ZB, wbody, (), unroll=False)

    @pl.when(c == 0)
    def _():
        pltpu.sync_copy(deg_acc.at[pl.ds(s * DPT, DPT)], zb_v.at[pl.ds(0, DPT)])
        pltpu.sync_copy(zb_v.at[pl.ds(0, DPT)], deg_hbm.at[pl.ds(s * DPT, DPT)])


_sc_counts = _sc_kernel(
    _sc_counts_body,
    out_type=(jax.ShapeDtypeStruct((CTOT,), jnp.float32),
              jax.ShapeDtypeStruct((CTOT,), jnp.float32),
              jax.ShapeDtypeStruct((DROWS,), jnp.float32)),
    scratch_types=[
        pltpu.VMEM((CH,), jnp.int32),
        pltpu.VMEM((CH,), jnp.int32),
        pltpu.VMEM((CH,), jnp.int32),
        pltpu.VMEM((CH,), jnp.int32),
        pltpu.VMEM((CH,), jnp.float32),
        pltpu.VMEM((ZB,), jnp.float32),
        pltpu.VMEM_SHARED((CTOT,), jnp.float32),
        pltpu.VMEM_SHARED((DROWS,), jnp.float32),
    ],
)


def _sc_msg_body(ent_hbm, rel_hbm, src_hbm, dst_hbm, typ_hbm, zrow_hbm,
                 out0_hbm, out1_hbm, src_v, dst_v, typ_v, rows_v, rows2_v,
                 acc, relsp, sem, sem2):
    c = lax.axis_index("c")
    s = lax.axis_index("s")
    pltpu.sync_copy(zrow_hbm, rows_v)

    def zbody(i, _):
        pltpu.sync_copy(rows_v, acc.at[pl.ds(s * S2_RPT + i * CH, CH)])
        return ()

    lax.fori_loop(0, S2_RPT // CH, zbody, (), unroll=False)

    @pl.when(s == 0)
    def _():
        pltpu.sync_copy(rel_hbm.at[pl.ds(0, 128)], rows2_v)
        pltpu.sync_copy(rows2_v, relsp.at[pl.ds(0, 128)])
        pltpu.sync_copy(rel_hbm.at[pl.ds(128, NT - 128)],
                        rows2_v.at[pl.ds(0, NT - 128)])
        pltpu.sync_copy(rows2_v.at[pl.ds(0, NT - 128)],
                        relsp.at[pl.ds(128, NT - 128)])

    plsc.subcore_barrier()

    w = s * NC + c
    n_chunks = PAD_E // NW // CH

    def body(i, _):
        base = (w * n_chunks + i) * CH
        pltpu.sync_copy(src_hbm.at[pl.ds(base, CH)], src_v)
        pltpu.sync_copy(dst_hbm.at[pl.ds(base, CH)], dst_v)
        pltpu.sync_copy(typ_hbm.at[pl.ds(base, CH)], typ_v)
        ent_cp = pltpu.async_copy(ent_hbm.at[src_v], rows_v, sem)
        rel_cp = pltpu.async_copy(relsp.at[typ_v], rows2_v, sem2)
        ent_cp.wait()
        pltpu.sync_copy(rows_v, acc.at[dst_v], add=True)
        rel_cp.wait()
        pltpu.sync_copy(rows2_v, acc.at[dst_v], add=True)
        return ()

    lax.fori_loop(0, n_chunks, body, (), unroll=False)
    plsc.subcore_barrier()

    def wbody(i, _):
        off = s * S2_RPT + i * CH
        pltpu.sync_copy(acc.at[pl.ds(off, CH)], rows_v)

        @pl.when(c == 0)
        def _():
            pltpu.sync_copy(rows_v, out0_hbm.at[pl.ds(off, CH)])

        @pl.when(c == 1)
        def _():
            pltpu.sync_copy(rows_v, out1_hbm.at[pl.ds(off, CH)])

        return ()

    lax.fori_loop(0, S2_RPT // CH, wbody, (), unroll=False)


_sc_msg = _sc_kernel(
    _sc_msg_body,
    out_type=(jax.ShapeDtypeStruct((S2_ROWS, H), jnp.float32),
              jax.ShapeDtypeStruct((S2_ROWS, H), jnp.float32)),
    scratch_types=[
        pltpu.VMEM((CH,), jnp.int32),
        pltpu.VMEM((CH,), jnp.int32),
        pltpu.VMEM((CH,), jnp.int32),
        pltpu.VMEM((CH, H), jnp.float32),
        pltpu.VMEM((CH, H), jnp.float32),
        pltpu.VMEM_SHARED((S2_ROWS, H), jnp.float32),
        pltpu.VMEM_SHARED((208, H), jnp.float32),
        pltpu.SemaphoreType.DMA,
        pltpu.SemaphoreType.DMA,
    ],
)


def _sc_gather_body(pre_hbm, rel_hbm, sub_hbm, relidx_hbm,
                    es_hbm, re_hbm, idx_v, rows_v, sem):
    c = lax.axis_index("c")
    s = lax.axis_index("s")
    w = s * NC + c
    n = B2 // NW
    base = w * n
    pltpu.sync_copy(sub_hbm.at[pl.ds(base, n)], idx_v)
    pltpu.async_copy(pre_hbm.at[idx_v], rows_v, sem).wait()
    pltpu.sync_copy(rows_v, es_hbm.at[pl.ds(base, n)])
    pltpu.sync_copy(relidx_hbm.at[pl.ds(base, n)], idx_v)
    pltpu.async_copy(rel_hbm.at[idx_v], rows_v, sem).wait()
    pltpu.sync_copy(rows_v, re_hbm.at[pl.ds(base, n)])


_sc_gather = _sc_kernel(
    _sc_gather_body,
    out_type=(jax.ShapeDtypeStruct((B2, H), jnp.float32),
              jax.ShapeDtypeStruct((B2, H), jnp.float32)),
    scratch_types=[
        pltpu.VMEM((B2 // NW,), jnp.int32),
        pltpu.VMEM((B2 // NW, H), jnp.float32),
        pltpu.SemaphoreType.DMA,
    ],
)


def _l2n(x):
    return x / jnp.maximum(jnp.sqrt(jnp.sum(x * x, axis=-1, keepdims=True)), 1e-12)


def _entnorm_body(x_ref, o_ref):
    o_ref[...] = _l2n(x_ref[...])


def _rel_body(c1_ref, ent_ref, rel_ref, wiht_ref, bih_ref, whht_ref, bhh_ref, o_ref):
    c1 = c1_ref[...]
    ent = ent_ref[...]
    rel_emb = rel_ref[...]
    s1 = jnp.dot(c1, ent, preferred_element_type=jnp.float32)
    cnt = jnp.sum(c1, axis=1, keepdims=True)
    x_mean = s1 / jnp.maximum(cnt, 1.0)
    x_in = jnp.concatenate([rel_emb, x_mean], axis=1)
    gi = jnp.dot(x_in, wiht_ref[...], preferred_element_type=jnp.float32) + bih_ref[...]
    gh = jnp.dot(rel_emb, whht_ref[...], preferred_element_type=jnp.float32) + bhh_ref[...]
    i_r, i_z, i_n = gi[:, :H], gi[:, H:2 * H], gi[:, 2 * H:]
    h_r, h_z, h_n = gh[:, :H], gh[:, H:2 * H], gh[:, 2 * H:]
    r_g = jax.nn.sigmoid(i_r + h_r)
    z_g = jax.nn.sigmoid(i_z + h_z)
    n_g = jnp.tanh(i_n + r_g * h_n)
    o_ref[...] = _l2n((1.0 - z_g) * n_g + z_g * rel_emb)


def _ent_update_body(ent_ref, ma_ref, mb_ref, deg_ref, wagg_ref,
                     wt_ref, bt_ref, o_ref):
    ent = ent_ref[...]
    agg = (ma_ref[...] + mb_ref[...]) / jnp.maximum(deg_ref[...], 1.0)
    curr = _l2n(jax.nn.relu(jnp.dot(agg, wagg_ref[...],
                                    preferred_element_type=jnp.float32)))
    tw = jax.nn.sigmoid(jnp.dot(ent, wt_ref[...],
                                preferred_element_type=jnp.float32) + bt_ref[...])
    o_ref[...] = _l2n(tw * curr + (1.0 - tw) * ent)


EBLK = 1000


def _decoder_body(es_ref, re_ref, sent_ref, w9_ref, cb_ref, fcw_ref, fcb_ref,
                  obj_ref, pre_ref, o_ref, feat_s, m_s, s_s, t_s):
    i = pl.program_id(0)

    @pl.when(i == 0)
    def _():
        zcol = jnp.zeros((B2, 1), jnp.float32)
        xs = []
        for k in range(3):
            for ci in range(3):
                x = (es_ref, re_ref, sent_ref)[ci][...]
                if k == 0:
                    x = jnp.concatenate([zcol, x[:, :H - 1]], axis=1)
                elif k == 2:
                    x = jnp.concatenate([x[:, 1:], zcol], axis=1)
                xs.append(x)
        facc = jnp.zeros((B2, H), jnp.float32)
        for o in range(CCH):
            y_o = w9_ref[o, 0] * xs[0]
            for j in range(1, 9):
                y_o = y_o + w9_ref[o, j] * xs[j]
            y_o = jax.nn.relu(y_o + cb_ref[0, o])
            facc = facc + jnp.dot(y_o, fcw_ref[o],
                                  preferred_element_type=jnp.float32)
        feat_s[...] = jax.nn.relu(facc + fcb_ref[...])
        m_s[...] = jnp.full((B2, 1), -1e30, jnp.float32)
        s_s[...] = jnp.zeros((B2, 1), jnp.float32)
        t_s[...] = jnp.zeros((B2, 1), jnp.float32)

    feat = feat_s[...]
    sc = lax.dot_general(feat, pre_ref[...], (((1,), (1,)), ((), ())),
                         preferred_element_type=jnp.float32)
    ids = i * EBLK + lax.broadcasted_iota(jnp.int32, (B2, EBLK), 1)
    hit = ids == obj_ref[...]
    t_s[...] = t_s[...] + jnp.sum(jnp.where(hit, sc, 0.0), axis=1, keepdims=True)
    m_old = m_s[...]
    m_new = jnp.maximum(m_old, jnp.max(sc, axis=1, keepdims=True))
    s_s[...] = s_s[...] * jnp.exp(m_old - m_new) + \
        jnp.sum(jnp.exp(sc - m_new), axis=1, keepdims=True)
    m_s[...] = m_new

    @pl.when(i == pl.num_programs(0) - 1)
    def _():
        o_ref[...] = jnp.mean(m_s[...] + jnp.log(s_s[...]) - t_s[...]).reshape(1, 1)


def kernel(dynamic_emb, emb_rel, W_ih, b_ih, W_hh, b_hh, W_agg, W_t, b_t,
           conv_w, conv_b, fc_w, fc_b, sentence_embeddings,
           edge_src, edge_dst, edge_type, sub, rel, obj):
    f32 = jnp.float32
    npad = PAD_E - E
    src_p = jnp.concatenate([edge_src.astype(jnp.int32),
                             jnp.zeros((npad,), jnp.int32)])
    typ_p = jnp.concatenate([edge_type.astype(jnp.int32),
                             jnp.full((npad,), NT, jnp.int32)])
    dst_p = jnp.concatenate([edge_dst.astype(jnp.int32),
                             jnp.full((npad,), NE, jnp.int32)])

    ent = pl.pallas_call(
        _entnorm_body,
        out_shape=jax.ShapeDtypeStruct((NE, H), f32),
    )(dynamic_emb[0])

    c1a, c1b, deg = _sc_counts(typ_p, src_p, dst_p, jnp.zeros((ZB,), f32))
    C1 = jnp.concatenate([c1a[:NR * NE].reshape(NR, NE),
                          c1b[:NR * NE].reshape(NR, NE)], axis=0)

    rel_new = pl.pallas_call(
        _rel_body,
        out_shape=jax.ShapeDtypeStruct((NT, H), f32),
    )(C1, ent, emb_rel[0], W_ih.T, b_ih[None, :], W_hh.T, b_hh[None, :])

    msg0, msg1 = _sc_msg(ent, rel_new, src_p, dst_p, typ_p,
                         jnp.zeros((CH, H), f32))

    nb = NE // EBLK
    pre_emb = pl.pallas_call(
        _ent_update_body,
        grid=(nb,),
        in_specs=[
            pl.BlockSpec((EBLK, H), lambda i: (i, 0)),
            pl.BlockSpec((EBLK, H), lambda i: (i, 0)),
            pl.BlockSpec((EBLK, H), lambda i: (i, 0)),
            pl.BlockSpec((EBLK, 1), lambda i: (i, 0)),
            pl.BlockSpec((H, H), lambda i: (0, 0)),
            pl.BlockSpec((H, H), lambda i: (0, 0)),
            pl.BlockSpec((1, H), lambda i: (0, 0)),
        ],
        out_specs=pl.BlockSpec((EBLK, H), lambda i: (i, 0)),
        out_shape=jax.ShapeDtypeStruct((NE, H), f32),
    )(ent, msg0[:NE], msg1[:NE], deg[:NE, None],
      W_agg, W_t, b_t[None, :])

    all_sub = jnp.concatenate([sub, obj]).astype(jnp.int32)
    all_rel = jnp.concatenate([rel, rel + NR]).astype(jnp.int32)
    all_obj = jnp.concatenate([obj, sub]).astype(jnp.int32)
    sent = jnp.concatenate([sentence_embeddings, sentence_embeddings], axis=0)

    e_s, r_e = _sc_gather(pre_emb, rel_new, all_sub, all_rel)

    w9 = conv_w.transpose(0, 2, 1).reshape(CCH, 9)
    fcw = fc_w.reshape(CCH, H, H)

    loss = pl.pallas_call(
        _decoder_body,
        grid=(nb,),
        in_specs=[
            pl.BlockSpec((B2, H), lambda i: (0, 0)),
            pl.BlockSpec((B2, H), lambda i: (0, 0)),
            pl.BlockSpec((B2, H), lambda i: (0, 0)),
            pl.BlockSpec((CCH, 9), lambda i: (0, 0)),
            pl.BlockSpec((1, CCH), lambda i: (0, 0)),
            pl.BlockSpec((CCH, H, H), lambda i: (0, 0, 0)),
            pl.BlockSpec((1, H), lambda i: (0, 0)),
            pl.BlockSpec((B2, 1), lambda i: (0, 0)),
            pl.BlockSpec((EBLK, H), lambda i: (i, 0)),
        ],
        out_specs=pl.BlockSpec((1, 1), lambda i: (0, 0)),
        out_shape=jax.ShapeDtypeStruct((1, 1), f32),
        scratch_shapes=[
            pltpu.VMEM((B2, H), f32),
            pltpu.VMEM((B2, 1), f32),
            pltpu.VMEM((B2, 1), f32),
            pltpu.VMEM((B2, 1), f32),
        ],
    )(e_s, r_e, sent, w9, conv_b[None, :], fcw, fc_b[None, :],
      all_obj[:, None], pre_emb)

    return loss[0, 0]


# speedup vs baseline: 4.9863x; 1.1507x over previous
import jax
import jax.numpy as jnp
from jax import lax
from jax.experimental import pallas as pl
from jax.experimental.pallas import tpu as pltpu
from jax.experimental.pallas import tpu_sc as plsc

NE = 10000
NR = 100
NT = 200
H = 128
E = 320000
B2 = 2048
CCH = 32

NC = 2
NS = 16
NW = NC * NS

CH = 128
PAD_E = 327680
NROW = PAD_E // CH
CTOT = 1024000
CPT = CTOT // NS
ZB = 8000
DROWS = 10240
DPT = DROWS // NS
S2_ROWS = 10240
S2_RPT = S2_ROWS // NS

_sc_mesh = plsc.VectorSubcoreMesh(core_axis_name="c", subcore_axis_name="s")


def _sc_kernel(body, out_type, scratch_types):
    return pl.kernel(body, out_type=out_type, mesh=_sc_mesh,
                     scratch_types=scratch_types)


GRP = 16


def _sc_counts_body(typ_hbm, src_hbm, dst_hbm, zeros_hbm,
                    c1a_hbm, c1b_hbm, deg_hbm, typ_v, src_v, dst_v, fidx_v,
                    ones_v, zb_v, acc, deg_acc, sem, semd):
    c = lax.axis_index("c")
    s = lax.axis_index("s")
    pltpu.sync_copy(zeros_hbm, zb_v)

    def zbody(i, _):
        pltpu.sync_copy(zb_v, acc.at[pl.ds(s * CPT + i * ZB, ZB)])
        return ()

    lax.fori_loop(0, CPT // ZB, zbody, (), unroll=False)

    @pl.when(c == 0)
    def _():
        pltpu.sync_copy(zb_v.at[pl.ds(0, DPT)], deg_acc.at[pl.ds(s * DPT, DPT)])

    for j in range(CH // 16):
        ones_v[pl.ds(j * 16, 16)] = jnp.ones((16,), jnp.float32)
    plsc.subcore_barrier()

    rows_pt = NROW // NS
    base_t = c * NR * NE

    def body(g, _):
        row0 = s * rows_pt + g * GRP
        pltpu.sync_copy(typ_hbm.at[pl.ds(row0, GRP)], typ_v)
        pltpu.sync_copy(src_hbm.at[pl.ds(row0, GRP)], src_v)

        @pl.when(c == 0)
        def _():
            pltpu.sync_copy(dst_hbm.at[pl.ds(row0, GRP)], dst_v)

        for r in range(GRP):
            for j in range(CH // 16):
                t = typ_v[r, pl.ds(j * 16, 16)]
                sr = src_v[r, pl.ds(j * 16, 16)]
                loc = t * NE + sr - base_t
                valid = (loc >= 0) & (loc < NR * NE)
                fidx_v[r, pl.ds(j * 16, 16)] = jnp.where(valid, loc, NR * NE + sr)
        descs = [pltpu.async_copy(ones_v, acc.at[fidx_v.at[r]], sem, add=True)
                 for r in range(GRP)]

        @pl.when(c == 0)
        def _():
            dd = [pltpu.async_copy(ones_v, deg_acc.at[dst_v.at[r]], semd,
                                   add=True) for r in range(GRP)]
            for d in dd:
                d.wait()

        for d in descs:
            d.wait()
        return ()

    lax.fori_loop(0, rows_pt // GRP, body, (), unroll=False)
    plsc.subcore_barrier()

    def wbody(i, _):
        off = s * CPT + i * ZB
        pltpu.sync_copy(acc.at[pl.ds(off, ZB)], zb_v)

        @pl.when(c == 0)
        def _():
            pltpu.sync_copy(zb_v, c1a_hbm.at[pl.ds(off, ZB)])

        @pl.when(c == 1)
        def _():
            pltpu.sync_copy(zb_v, c1b_hbm.at[pl.ds(off, ZB)])

        return ()

    lax.fori_loop(0, CPT // ZB, wbody, (), unroll=False)

    @pl.when(c == 0)
    def _():
        pltpu.sync_copy(deg_acc.at[pl.ds(s * DPT, DPT)], zb_v.at[pl.ds(0, DPT)])
        pltpu.sync_copy(zb_v.at[pl.ds(0, DPT)], deg_hbm.at[pl.ds(s * DPT, DPT)])


_sc_counts = _sc_kernel(
    _sc_counts_body,
    out_type=(jax.ShapeDtypeStruct((CTOT,), jnp.float32),
              jax.ShapeDtypeStruct((CTOT,), jnp.float32),
              jax.ShapeDtypeStruct((DROWS,), jnp.float32)),
    scratch_types=[
        pltpu.VMEM((GRP, CH), jnp.int32),
        pltpu.VMEM((GRP, CH), jnp.int32),
        pltpu.VMEM((GRP, CH), jnp.int32),
        pltpu.VMEM((GRP, CH), jnp.int32),
        pltpu.VMEM((CH,), jnp.float32),
        pltpu.VMEM((ZB,), jnp.float32),
        pltpu.VMEM_SHARED((CTOT,), jnp.float32),
        pltpu.VMEM_SHARED((DROWS,), jnp.float32),
        pltpu.SemaphoreType.DMA,
        pltpu.SemaphoreType.DMA,
    ],
)


MGRP = 8


def _sc_msg_body(ent_hbm, rel_hbm, src_hbm, dst_hbm, typ_hbm, zrow_hbm,
                 out0_hbm, out1_hbm, src2, dst2, typ2, eb0, rb0,
                 acc, relsp, se0, sr0):
    c = lax.axis_index("c")
    s = lax.axis_index("s")
    pltpu.sync_copy(zrow_hbm, eb0)

    def zbody(i, _):
        pltpu.sync_copy(eb0, acc.at[pl.ds(s * S2_RPT + i * CH, CH)])
        return ()

    lax.fori_loop(0, S2_RPT // CH, zbody, (), unroll=False)

    @pl.when(s == 0)
    def _():
        pltpu.sync_copy(rel_hbm.at[pl.ds(0, 128)], rb0)
        pltpu.sync_copy(rb0, relsp.at[pl.ds(0, 128)])
        pltpu.sync_copy(rel_hbm.at[pl.ds(128, NT - 128)],
                        rb0.at[pl.ds(0, NT - 128)])
        pltpu.sync_copy(rb0.at[pl.ds(0, NT - 128)],
                        relsp.at[pl.ds(128, NT - 128)])

    plsc.subcore_barrier()

    w = s * NC + c
    rows_pw = NROW // NW
    ngrp = rows_pw // MGRP
    r0 = w * rows_pw

    def load_grp(g):
        pltpu.sync_copy(src_hbm.at[pl.ds(r0 + g * MGRP, MGRP)], src2)
        pltpu.sync_copy(dst_hbm.at[pl.ds(r0 + g * MGRP, MGRP)], dst2)
        pltpu.sync_copy(typ_hbm.at[pl.ds(r0 + g * MGRP, MGRP)], typ2)

    load_grp(0)
    pltpu.async_copy(ent_hbm.at[src2.at[0]], eb0, se0)
    pltpu.async_copy(relsp.at[typ2.at[0]], rb0, sr0)

    def body(g, _):
        for k in range(MGRP):
            pltpu.make_async_copy(zrow_hbm, eb0, se0).wait()
            pltpu.sync_copy(eb0, acc.at[dst2.at[k]], add=True)
            if k < MGRP - 1:
                pltpu.async_copy(ent_hbm.at[src2.at[k + 1]], eb0, se0)
                pltpu.make_async_copy(zrow_hbm, rb0, sr0).wait()
                pltpu.sync_copy(rb0, acc.at[dst2.at[k]], add=True)
                pltpu.async_copy(relsp.at[typ2.at[k + 1]], rb0, sr0)
            else:
                pltpu.make_async_copy(zrow_hbm, rb0, sr0).wait()
                pltpu.sync_copy(rb0, acc.at[dst2.at[k]], add=True)

                @pl.when(g < ngrp - 1)
                def _():
                    load_grp(g + 1)
                    pltpu.async_copy(ent_hbm.at[src2.at[0]], eb0, se0)
                    pltpu.async_copy(relsp.at[typ2.at[0]], rb0, sr0)

        return ()

    lax.fori_loop(0, ngrp, body, (), unroll=False)
    plsc.subcore_barrier()

    def wbody(i, _):
        off = s * S2_RPT + i * CH
        pltpu.sync_copy(acc.at[pl.ds(off, CH)], eb0)

        @pl.when(c == 0)
        def _():
            pltpu.sync_copy(eb0, out0_hbm.at[pl.ds(off, CH)])

        @pl.when(c == 1)
        def _():
            pltpu.sync_copy(eb0, out1_hbm.at[pl.ds(off, CH)])

        return ()

    lax.fori_loop(0, S2_RPT // CH, wbody, (), unroll=False)


_sc_msg = _sc_kernel(
    _sc_msg_body,
    out_type=(jax.ShapeDtypeStruct((S2_ROWS, H), jnp.float32),
              jax.ShapeDtypeStruct((S2_ROWS, H), jnp.float32)),
    scratch_types=[
        pltpu.VMEM((MGRP, CH), jnp.int32),
        pltpu.VMEM((MGRP, CH), jnp.int32),
        pltpu.VMEM((MGRP, CH), jnp.int32),
        pltpu.VMEM((CH, H), jnp.float32),
        pltpu.VMEM((CH, H), jnp.float32),
        pltpu.VMEM_SHARED((S2_ROWS, H), jnp.float32),
        pltpu.VMEM_SHARED((208, H), jnp.float32),
        pltpu.SemaphoreType.DMA,
        pltpu.SemaphoreType.DMA,
    ],
)


def _sc_gather_body(pre_hbm, rel_hbm, sub_hbm, relidx_hbm,
                    es_hbm, re_hbm, idx_v, rows_v, sem):
    c = lax.axis_index("c")
    s = lax.axis_index("s")
    w = s * NC + c
    n = B2 // NW
    base = w * n
    pltpu.sync_copy(sub_hbm.at[pl.ds(base, n)], idx_v)
    pltpu.async_copy(pre_hbm.at[idx_v], rows_v, sem).wait()
    pltpu.sync_copy(rows_v, es_hbm.at[pl.ds(base, n)])
    pltpu.sync_copy(relidx_hbm.at[pl.ds(base, n)], idx_v)
    pltpu.async_copy(rel_hbm.at[idx_v], rows_v, sem).wait()
    pltpu.sync_copy(rows_v, re_hbm.at[pl.ds(base, n)])


_sc_gather = _sc_kernel(
    _sc_gather_body,
    out_type=(jax.ShapeDtypeStruct((B2, H), jnp.float32),
              jax.ShapeDtypeStruct((B2, H), jnp.float32)),
    scratch_types=[
        pltpu.VMEM((B2 // NW,), jnp.int32),
        pltpu.VMEM((B2 // NW, H), jnp.float32),
        pltpu.SemaphoreType.DMA,
    ],
)


def _l2n(x):
    return x / jnp.maximum(jnp.sqrt(jnp.sum(x * x, axis=-1, keepdims=True)), 1e-12)


def _entnorm_body(x_ref, o_ref):
    o_ref[...] = _l2n(x_ref[...])


def _rel_body(c1_ref, ent_ref, rel_ref, wiht_ref, bih_ref, whht_ref, bhh_ref, o_ref):
    c1 = c1_ref[...]
    ent = ent_ref[...]
    rel_emb = rel_ref[...]
    s1 = jnp.dot(c1, ent, preferred_element_type=jnp.float32)
    cnt = jnp.sum(c1, axis=1, keepdims=True)
    x_mean = s1 / jnp.maximum(cnt, 1.0)
    x_in = jnp.concatenate([rel_emb, x_mean], axis=1)
    gi = jnp.dot(x_in, wiht_ref[...], preferred_element_type=jnp.float32) + bih_ref[...]
    gh = jnp.dot(rel_emb, whht_ref[...], preferred_element_type=jnp.float32) + bhh_ref[...]
    i_r, i_z, i_n = gi[:, :H], gi[:, H:2 * H], gi[:, 2 * H:]
    h_r, h_z, h_n = gh[:, :H], gh[:, H:2 * H], gh[:, 2 * H:]
    r_g = jax.nn.sigmoid(i_r + h_r)
    z_g = jax.nn.sigmoid(i_z + h_z)
    n_g = jnp.tanh(i_n + r_g * h_n)
    o_ref[...] = _l2n((1.0 - z_g) * n_g + z_g * rel_emb)


def _ent_update_body(ent_ref, ma_ref, mb_ref, deg_ref, wagg_ref,
                     wt_ref, bt_ref, o_ref):
    ent = ent_ref[...]
    agg = (ma_ref[...] + mb_ref[...]) / jnp.maximum(deg_ref[...], 1.0)
    curr = _l2n(jax.nn.relu(jnp.dot(agg, wagg_ref[...],
                                    preferred_element_type=jnp.float32)))
    tw = jax.nn.sigmoid(jnp.dot(ent, wt_ref[...],
                                preferred_element_type=jnp.float32) + bt_ref[...])
    o_ref[...] = _l2n(tw * curr + (1.0 - tw) * ent)


EBLK = 1000


def _decoder_body(es_ref, re_ref, sent_ref, w9_ref, cb_ref, fcw_ref, fcb_ref,
                  obj_ref, pre_ref, o_ref, feat_s, m_s, s_s, t_s):
    i = pl.program_id(0)

    @pl.when(i == 0)
    def _():
        zcol = jnp.zeros((B2, 1), jnp.float32)
        xs = []
        for k in range(3):
            for ci in range(3):
                x = (es_ref, re_ref, sent_ref)[ci][...]
                if k == 0:
                    x = jnp.concatenate([zcol, x[:, :H - 1]], axis=1)
                elif k == 2:
                    x = jnp.concatenate([x[:, 1:], zcol], axis=1)
                xs.append(x)
        facc = jnp.zeros((B2, H), jnp.float32)
        for o in range(CCH):
            y_o = w9_ref[o, 0] * xs[0]
            for j in range(1, 9):
                y_o = y_o + w9_ref[o, j] * xs[j]
            y_o = jax.nn.relu(y_o + cb_ref[0, o])
            facc = facc + jnp.dot(y_o, fcw_ref[o],
                                  preferred_element_type=jnp.float32)
        feat_s[...] = jax.nn.relu(facc + fcb_ref[...])
        m_s[...] = jnp.full((B2, 1), -1e30, jnp.float32)
        s_s[...] = jnp.zeros((B2, 1), jnp.float32)
        t_s[...] = jnp.zeros((B2, 1), jnp.float32)

    feat = feat_s[...]
    sc = lax.dot_general(feat, pre_ref[...], (((1,), (1,)), ((), ())),
                         preferred_element_type=jnp.float32)
    ids = i * EBLK + lax.broadcasted_iota(jnp.int32, (B2, EBLK), 1)
    hit = ids == obj_ref[...]
    t_s[...] = t_s[...] + jnp.sum(jnp.where(hit, sc, 0.0), axis=1, keepdims=True)
    m_old = m_s[...]
    m_new = jnp.maximum(m_old, jnp.max(sc, axis=1, keepdims=True))
    s_s[...] = s_s[...] * jnp.exp(m_old - m_new) + \
        jnp.sum(jnp.exp(sc - m_new), axis=1, keepdims=True)
    m_s[...] = m_new

    @pl.when(i == pl.num_programs(0) - 1)
    def _():
        o_ref[...] = jnp.mean(m_s[...] + jnp.log(s_s[...]) - t_s[...]).reshape(1, 1)


def kernel(dynamic_emb, emb_rel, W_ih, b_ih, W_hh, b_hh, W_agg, W_t, b_t,
           conv_w, conv_b, fc_w, fc_b, sentence_embeddings,
           edge_src, edge_dst, edge_type, sub, rel, obj):
    f32 = jnp.float32
    npad = PAD_E - E
    src_p = jnp.concatenate([edge_src.astype(jnp.int32),
                             jnp.zeros((npad,), jnp.int32)])
    typ_p = jnp.concatenate([edge_type.astype(jnp.int32),
                             jnp.full((npad,), NT, jnp.int32)])
    dst_p = jnp.concatenate([edge_dst.astype(jnp.int32),
                             jnp.full((npad,), NE, jnp.int32)])

    ent = pl.pallas_call(
        _entnorm_body,
        out_shape=jax.ShapeDtypeStruct((NE, H), f32),
    )(dynamic_emb[0])

    src_p = src_p.reshape(NROW, CH)
    typ_p = typ_p.reshape(NROW, CH)
    dst_p = dst_p.reshape(NROW, CH)

    c1a, c1b, deg = _sc_counts(typ_p, src_p, dst_p, jnp.zeros((ZB,), f32))
    C1 = jnp.concatenate([c1a[:NR * NE].reshape(NR, NE),
                          c1b[:NR * NE].reshape(NR, NE)], axis=0)

    rel_new = pl.pallas_call(
        _rel_body,
        out_shape=jax.ShapeDtypeStruct((NT, H), f32),
    )(C1, ent, emb_rel[0], W_ih.T, b_ih[None, :], W_hh.T, b_hh[None, :])

    msg0, msg1 = _sc_msg(ent, rel_new, src_p, dst_p, typ_p,
                         jnp.zeros((CH, H), f32))

    nb = NE // EBLK
    pre_emb = pl.pallas_call(
        _ent_update_body,
        grid=(nb,),
        in_specs=[
            pl.BlockSpec((EBLK, H), lambda i: (i, 0)),
            pl.BlockSpec((EBLK, H), lambda i: (i, 0)),
            pl.BlockSpec((EBLK, H), lambda i: (i, 0)),
            pl.BlockSpec((EBLK, 1), lambda i: (i, 0)),
            pl.BlockSpec((H, H), lambda i: (0, 0)),
            pl.BlockSpec((H, H), lambda i: (0, 0)),
            pl.BlockSpec((1, H), lambda i: (0, 0)),
        ],
        out_specs=pl.BlockSpec((EBLK, H), lambda i: (i, 0)),
        out_shape=jax.ShapeDtypeStruct((NE, H), f32),
    )(ent, msg0[:NE], msg1[:NE], deg[:NE, None],
      W_agg, W_t, b_t[None, :])

    all_sub = jnp.concatenate([sub, obj]).astype(jnp.int32)
    all_rel = jnp.concatenate([rel, rel + NR]).astype(jnp.int32)
    all_obj = jnp.concatenate([obj, sub]).astype(jnp.int32)
    sent = jnp.concatenate([sentence_embeddings, sentence_embeddings], axis=0)

    e_s, r_e = _sc_gather(pre_emb, rel_new, all_sub, all_rel)

    w9 = conv_w.transpose(0, 2, 1).reshape(CCH, 9)
    fcw = fc_w.reshape(CCH, H, H)

    loss = pl.pallas_call(
        _decoder_body,
        grid=(nb,),
        in_specs=[
            pl.BlockSpec((B2, H), lambda i: (0, 0)),
            pl.BlockSpec((B2, H), lambda i: (0, 0)),
            pl.BlockSpec((B2, H), lambda i: (0, 0)),
            pl.BlockSpec((CCH, 9), lambda i: (0, 0)),
            pl.BlockSpec((1, CCH), lambda i: (0, 0)),
            pl.BlockSpec((CCH, H, H), lambda i: (0, 0, 0)),
            pl.BlockSpec((1, H), lambda i: (0, 0)),
            pl.BlockSpec((B2, 1), lambda i: (0, 0)),
            pl.BlockSpec((EBLK, H), lambda i: (i, 0)),
        ],
        out_specs=pl.BlockSpec((1, 1), lambda i: (0, 0)),
        out_shape=jax.ShapeDtypeStruct((1, 1), f32),
        scratch_shapes=[
            pltpu.VMEM((B2, H), f32),
            pltpu.VMEM((B2, 1), f32),
            pltpu.VMEM((B2, 1), f32),
            pltpu.VMEM((B2, 1), f32),
        ],
    )(e_s, r_e, sent, w9, conv_b[None, :], fcw, fc_b[None, :],
      all_obj[:, None], pre_emb)

    return loss[0, 0]
